# baseline (speedup 1.0000x reference)
import functools

import jax
import jax.numpy as jnp
from jax import lax
from jax.experimental import pallas as pl
from jax.experimental.pallas import tpu as pltpu
from jax.experimental.pallas import tpu_sc as plsc

N = 10000
E = 320000
F_IN = 128
F_HID = 256
F_OUT = 64

NC = 2
NS = 16
NW = NC * NS
L = 16

C = 112
EP = E // NW
K = 91
EPAD = K * C
NPAD = 10240
ZR = NPAD // NS


def _make_spmm(cf):
    mesh = plsc.VectorSubcoreMesh(
        core_axis_name="c", subcore_axis_name="s",
        num_cores=NC, num_subcores=NS)

    @functools.partial(
        pl.kernel,
        out_type=jax.ShapeDtypeStruct((NC, NPAD, cf), jnp.float32),
        mesh=mesh,
        scratch_types=[
            pltpu.VMEM((C, cf), jnp.float32),
            pltpu.VMEM((C, cf), jnp.float32),
            pltpu.VMEM((C, cf), jnp.float32),
            pltpu.VMEM((3, C), jnp.int32),
            pltpu.VMEM((3, C), jnp.int32),
            pltpu.VMEM((3, C), jnp.int32),
            pltpu.VMEM((3, C), jnp.int32),
            pltpu.VMEM((3, C), jnp.int32),
            pltpu.VMEM((3, C), jnp.int32),
            pltpu.VMEM_SHARED((NPAD, cf), jnp.float32),
            pltpu.SemaphoreType.DMA,
            pltpu.SemaphoreType.DMA,
            pltpu.SemaphoreType.DMA,
            pltpu.SemaphoreType.DMA,
            pltpu.SemaphoreType.DMA,
            pltpu.SemaphoreType.DMA,
            pltpu.SemaphoreType.DMA,
            pltpu.SemaphoreType.DMA,
            pltpu.SemaphoreType.DMA,
            pltpu.SemaphoreType.DMA,
            pltpu.SemaphoreType.DMA,
            pltpu.SemaphoreType.DMA,
        ],
    )
    def spmm(x_hbm, idxw_h, z_h, out_h,
             r0, r1, r2, x0, x1, x2, x3, x4, x5, acc_s,
             g0, g1, g2, s0, s1, s2, i0, i1, i2, i3, i4, i5):
        ci = lax.axis_index("c")
        si = lax.axis_index("s")
        wid = si * NC + ci
        rows = (r0, r1, r2)
        ix = (x0, x1, x2, x3, x4, x5)
        gsem = (g0, g1, g2)
        ssem = (s0, s1, s2)
        isem = (i0, i1, i2, i3, i4, i5)

        pltpu.sync_copy(z_h.at[pl.ds(si * ZR, ZR)],
                        acc_s.at[pl.ds(si * ZR, ZR)])
        plsc.subcore_barrier()

        def start_idxw(k, q):
            pltpu.async_copy(idxw_h.at[wid, k], ix[q], isem[q])

        def wait_idxw(q):
            pltpu.make_async_copy(idxw_h.at[wid, 0], ix[q], isem[q]).wait()

        def start_gather(b, q):
            pltpu.async_copy(x_hbm.at[ix[q].at[0]], rows[b], gsem[b])

        def wait_gather(b):
            pltpu.make_async_copy(x_hbm.at[ix[0].at[0]], rows[b],
                                  gsem[b]).wait()

        def start_scatter(b, q):
            pltpu.async_copy(rows[b], acc_s.at[ix[q].at[1]], ssem[b],
                             add=True)

        def wait_scatter(b):
            pltpu.make_async_copy(rows[b], acc_s.at[ix[0].at[1]],
                                  ssem[b]).wait()

        def scale(b, q):
            buf = rows[b]

            def edge_group(g, c2):
                wvec = lax.bitcast_convert_type(
                    ix[q][2, pl.ds(g * L, L)], jnp.float32)
                for t in range(L):
                    swv = jnp.full((L,), wvec[t], dtype=jnp.float32)
                    i = g * L + t
                    for j in range(cf // L):
                        sl = pl.ds(j * L, L)
                        buf[i, sl] = buf[i, sl] * swv
                return c2
            lax.fori_loop(0, C // L, edge_group, 0)

        start_idxw(0, 0)
        start_idxw(1, 1)
        start_idxw(2, 2)
        wait_idxw(0)
        start_gather(0, 0)
        wait_idxw(1)
        start_gather(1, 1)

        U = 6

        def block(p, carry):
            k0 = U * p
            for t in range(U):
                k = k0 + t
                b = t % 3
                q = t % 6
                b2 = (t + 2) % 3
                q2 = (t + 2) % 6
                q3 = (t + 3) % 6
                wait_gather(b)
                scale(b, q)
                if t == 0:
                    @pl.when(k >= 1)
                    def _():
                        wait_scatter(b2)
                else:
                    wait_scatter(b2)
                start_scatter(b, q)

                @pl.when(k + 3 <= K - 1)
                def _():
                    start_idxw(k + 3, q3)

                @pl.when(k + 2 <= K - 1)
                def _():
                    wait_idxw(q2)
                    start_gather(b2, q2)
            return carry

        lax.fori_loop(0, (K - 1) // U, block, 0)
        wait_gather(0)
        scale(0, 0)
        wait_scatter(2)
        start_scatter(0, 0)
        wait_scatter(0)
        plsc.subcore_barrier()

        pltpu.sync_copy(acc_s.at[pl.ds(si * ZR, ZR)],
                        out_h.at[ci, pl.ds(si * ZR, ZR)])

    return spmm


RB = 1000


def _layer0_mm(agg, W0, b0):
    def body(a_ref, b_ref, w_ref, bias_ref, oa_ref, ob_ref):
        agg_ = a_ref[...] + b_ref[...]
        h = jnp.dot(agg_, w_ref[...], preferred_element_type=jnp.float32)
        h = jnp.maximum(h + bias_ref[...], 0.0)
        oa_ref[...] = h[:, :F_IN]
        ob_ref[...] = h[:, F_IN:]

    return pl.pallas_call(
        body,
        grid=(N // RB,),
        in_specs=[
            pl.BlockSpec((None, RB, F_IN), lambda i: (0, i, 0)),
            pl.BlockSpec((None, RB, F_IN), lambda i: (1, i, 0)),
            pl.BlockSpec((F_IN, F_HID), lambda i: (0, 0)),
            pl.BlockSpec((1, F_HID), lambda i: (0, 0)),
        ],
        out_specs=[
            pl.BlockSpec((RB, F_IN), lambda i: (i, 0)),
            pl.BlockSpec((RB, F_IN), lambda i: (i, 0)),
        ],
        out_shape=[jax.ShapeDtypeStruct((N, F_IN), jnp.float32)] * 2,
    )(agg, agg, W0, b0)


def _layer12_mm(o1a, o1b, W1, b1, W2):
    def body(a0, a1, c0, c1, w1_ref, bias_ref, w2_ref, y_ref):
        agg0 = a0[...] + a1[...]
        agg1 = c0[...] + c1[...]
        w1 = w1_ref[...]
        h = jnp.dot(agg0, w1[:F_IN], preferred_element_type=jnp.float32)
        h = h + jnp.dot(agg1, w1[F_IN:], preferred_element_type=jnp.float32)
        h = jnp.maximum(h + bias_ref[...], 0.0)
        y = jnp.dot(h, w2_ref[...], preferred_element_type=jnp.float32)
        y_ref[...] = jnp.concatenate(
            [y, jnp.zeros((RB, F_IN - F_OUT), jnp.float32)], axis=1)

    return pl.pallas_call(
        body,
        grid=(N // RB,),
        in_specs=[
            pl.BlockSpec((None, RB, F_IN), lambda i: (0, i, 0)),
            pl.BlockSpec((None, RB, F_IN), lambda i: (1, i, 0)),
            pl.BlockSpec((None, RB, F_IN), lambda i: (0, i, 0)),
            pl.BlockSpec((None, RB, F_IN), lambda i: (1, i, 0)),
            pl.BlockSpec((F_HID, F_HID), lambda i: (0, 0)),
            pl.BlockSpec((1, F_HID), lambda i: (0, 0)),
            pl.BlockSpec((F_HID, F_OUT), lambda i: (0, 0)),
        ],
        out_specs=pl.BlockSpec((RB, F_IN), lambda i: (i, 0)),
        out_shape=jax.ShapeDtypeStruct((N, F_IN), jnp.float32),
    )(o1a, o1a, o1b, o1b, W1, b1, W2)


def _layer2_bias(o2, b2):
    def body(a_ref, b_ref, bias_ref, o_ref):
        o_ref[...] = a_ref[..., :F_OUT] + b_ref[..., :F_OUT] + bias_ref[...]

    return pl.pallas_call(
        body,
        grid=(N // RB,),
        in_specs=[
            pl.BlockSpec((None, RB, F_IN), lambda i: (0, i, 0)),
            pl.BlockSpec((None, RB, F_IN), lambda i: (1, i, 0)),
            pl.BlockSpec((1, F_OUT), lambda i: (0, 0)),
        ],
        out_specs=pl.BlockSpec((RB, F_OUT), lambda i: (i, 0)),
        out_shape=jax.ShapeDtypeStruct((N, F_OUT), jnp.float32),
    )(o2, o2, b2)


def kernel(features, edge_index, edge_weight, W0, b0, W1, b1, W2, b2):
    src = edge_index[0]
    dst = edge_index[1]
    pad = ((0, 0), (0, EPAD - EP))
    srcp = jnp.pad(src.reshape(NW, EP), pad).reshape(NW, K, C)
    padvals = N + (jnp.arange(EPAD - EP, dtype=jnp.int32) % (NPAD - N))
    dstp = jnp.pad(dst.reshape(NW, EP), pad)
    dstp = dstp.at[:, EP:].set(padvals[None, :]).reshape(NW, K, C)
    wbits = jax.lax.bitcast_convert_type(
        jnp.pad(edge_weight.reshape(NW, EP), pad), jnp.int32).reshape(NW, K, C)
    idxw = jnp.stack([srcp, dstp, wbits], axis=2)
    z128 = jnp.zeros((NPAD, F_IN), jnp.float32)

    spmm128 = _make_spmm(F_IN)

    agg0 = spmm128(features, idxw, z128)
    h0a, h0b = _layer0_mm(agg0, W0, b0.reshape(1, F_HID))
    o1a = spmm128(h0a, idxw, z128)
    o1b = spmm128(h0b, idxw, z128)
    y2 = _layer12_mm(o1a, o1b, W1, b1.reshape(1, F_HID), W2)
    o2 = spmm128(y2, idxw, z128)
    return _layer2_bias(o2, b2.reshape(1, F_OUT))


# speedup vs baseline: 1.3024x; 1.3024x over previous
import functools

import jax
import jax.numpy as jnp
from jax import lax
from jax.experimental import pallas as pl
from jax.experimental.pallas import tpu as pltpu
from jax.experimental.pallas import tpu_sc as plsc

N = 10000
E = 320000
F_IN = 128
F_HID = 256
F_OUT = 64

NC = 2
NS = 16
NW = NC * NS
L = 16

C = 128
EP = E // NW
K = 79
EPAD = K * C
NPAD = 10240
ZR = NPAD // NS


def _make_spmm(cf):
    mesh = plsc.VectorSubcoreMesh(
        core_axis_name="c", subcore_axis_name="s",
        num_cores=NC, num_subcores=NS)

    @functools.partial(
        pl.kernel,
        out_type=jax.ShapeDtypeStruct((NC, NPAD, cf), jnp.float32),
        mesh=mesh,
        scratch_types=[
            pltpu.VMEM((C, cf), jnp.float32),
            pltpu.VMEM((C, cf), jnp.float32),
            pltpu.VMEM((3, C), jnp.int32),
            pltpu.VMEM((3, C), jnp.int32),
            pltpu.VMEM((3, C), jnp.int32),
            pltpu.VMEM_SHARED((NPAD, cf), jnp.float32),
            pltpu.SemaphoreType.DMA,
            pltpu.SemaphoreType.DMA,
            pltpu.SemaphoreType.DMA,
            pltpu.SemaphoreType.DMA,
            pltpu.SemaphoreType.DMA,
        ],
    )
    def spmm(x_hbm, idxw_h, out_h,
             r0, r1, x0, x1, x2, acc_s,
             g0, g1, i0, i1, i2):
        ci = lax.axis_index("c")
        si = lax.axis_index("s")
        wid = si * NC + ci
        rows = (r0, r1)
        ix = (x0, x1, x2)
        gsem = (g0, g1)
        isem = (i0, i1, i2)

        def zrow(i, c2):
            for j in range(cf // L):
                r0[i, pl.ds(j * L, L)] = jnp.zeros((L,), jnp.float32)
            return c2
        lax.fori_loop(0, C, zrow, 0)
        for v in range(ZR // C):
            pltpu.sync_copy(r0, acc_s.at[pl.ds(si * ZR + v * C, C)])
        plsc.subcore_barrier()

        def start_idxw(k, q):
            pltpu.async_copy(idxw_h.at[wid, k], ix[q], isem[q])

        def wait_idxw(q):
            pltpu.make_async_copy(idxw_h.at[wid, 0], ix[q], isem[q]).wait()

        def start_gather(b, q):
            pltpu.async_copy(x_hbm.at[ix[q].at[0]], rows[b], gsem[b])

        def wait_gather(b):
            pltpu.make_async_copy(x_hbm.at[ix[0].at[0]], rows[b],
                                  gsem[b]).wait()

        def sync_scatter(b, q):
            pltpu.sync_copy(rows[b], acc_s.at[ix[q].at[1]], add=True)

        def scale(b, q):
            buf = rows[b]

            @plsc.parallel_loop(0, C // L, unroll=2)
            def edge_group(g):
                wvec = lax.bitcast_convert_type(
                    ix[q][2, pl.ds(g * L, L)], jnp.float32)
                for t in range(L):
                    swv = jnp.full((L,), wvec[t], dtype=jnp.float32)
                    i = g * L + t
                    for j in range(cf // L):
                        sl = pl.ds(j * L, L)
                        buf[i, sl] = buf[i, sl] * swv

        start_idxw(0, 0)
        start_idxw(1, 1)
        start_idxw(2, 2)
        wait_idxw(0)
        start_gather(0, 0)
        wait_idxw(1)
        start_gather(1, 1)

        U = 6

        def block(p, carry):
            k0 = U * p
            for t in range(U):
                k = k0 + t
                b = t % 2
                q = t % 3
                q2 = (t + 2) % 3
                wait_gather(b)
                scale(b, q)
                sync_scatter(b, q)

                @pl.when(k + 3 <= K - 1)
                def _():
                    start_idxw(k + 3, q)

                @pl.when(k + 2 <= K - 1)
                def _():
                    wait_idxw(q2)
                    start_gather(b, q2)
            return carry

        lax.fori_loop(0, (K - 1) // U, block, 0)
        wait_gather(0)
        scale(0, 0)
        sync_scatter(0, 0)
        plsc.subcore_barrier()

        pltpu.sync_copy(acc_s.at[pl.ds(si * ZR, ZR)],
                        out_h.at[ci, pl.ds(si * ZR, ZR)])

    return spmm


RB = 1000


def _layer0_mm(agg, W0, b0):
    def body(a_ref, b_ref, w_ref, bias_ref, oa_ref, ob_ref):
        agg_ = a_ref[...] + b_ref[...]
        h = jnp.dot(agg_, w_ref[...], preferred_element_type=jnp.float32)
        h = jnp.maximum(h + bias_ref[...], 0.0)
        oa_ref[...] = h[:, :F_IN]
        ob_ref[...] = h[:, F_IN:]

    return pl.pallas_call(
        body,
        grid=(N // RB,),
        in_specs=[
            pl.BlockSpec((None, RB, F_IN), lambda i: (0, i, 0)),
            pl.BlockSpec((None, RB, F_IN), lambda i: (1, i, 0)),
            pl.BlockSpec((F_IN, F_HID), lambda i: (0, 0)),
            pl.BlockSpec((1, F_HID), lambda i: (0, 0)),
        ],
        out_specs=[
            pl.BlockSpec((RB, F_IN), lambda i: (i, 0)),
            pl.BlockSpec((RB, F_IN), lambda i: (i, 0)),
        ],
        out_shape=[jax.ShapeDtypeStruct((N, F_IN), jnp.float32)] * 2,
    )(agg, agg, W0, b0)


def _layer12_mm(o1a, o1b, W1, b1, W2):
    def body(a0, a1, c0, c1, w1_ref, bias_ref, w2_ref, y_ref):
        agg0 = a0[...] + a1[...]
        agg1 = c0[...] + c1[...]
        w1 = w1_ref[...]
        h = jnp.dot(agg0, w1[:F_IN], preferred_element_type=jnp.float32)
        h = h + jnp.dot(agg1, w1[F_IN:], preferred_element_type=jnp.float32)
        h = jnp.maximum(h + bias_ref[...], 0.0)
        y = jnp.dot(h, w2_ref[...], preferred_element_type=jnp.float32)
        y_ref[...] = jnp.concatenate(
            [y, jnp.zeros((RB, F_IN - F_OUT), jnp.float32)], axis=1)

    return pl.pallas_call(
        body,
        grid=(N // RB,),
        in_specs=[
            pl.BlockSpec((None, RB, F_IN), lambda i: (0, i, 0)),
            pl.BlockSpec((None, RB, F_IN), lambda i: (1, i, 0)),
            pl.BlockSpec((None, RB, F_IN), lambda i: (0, i, 0)),
            pl.BlockSpec((None, RB, F_IN), lambda i: (1, i, 0)),
            pl.BlockSpec((F_HID, F_HID), lambda i: (0, 0)),
            pl.BlockSpec((1, F_HID), lambda i: (0, 0)),
            pl.BlockSpec((F_HID, F_OUT), lambda i: (0, 0)),
        ],
        out_specs=pl.BlockSpec((RB, F_IN), lambda i: (i, 0)),
        out_shape=jax.ShapeDtypeStruct((N, F_IN), jnp.float32),
    )(o1a, o1a, o1b, o1b, W1, b1, W2)


def _layer2_bias(o2, b2):
    def body(a_ref, b_ref, bias_ref, o_ref):
        o_ref[...] = a_ref[..., :F_OUT] + b_ref[..., :F_OUT] + bias_ref[...]

    return pl.pallas_call(
        body,
        grid=(N // RB,),
        in_specs=[
            pl.BlockSpec((None, RB, F_IN), lambda i: (0, i, 0)),
            pl.BlockSpec((None, RB, F_IN), lambda i: (1, i, 0)),
            pl.BlockSpec((1, F_OUT), lambda i: (0, 0)),
        ],
        out_specs=pl.BlockSpec((RB, F_OUT), lambda i: (i, 0)),
        out_shape=jax.ShapeDtypeStruct((N, F_OUT), jnp.float32),
    )(o2, o2, b2)


def kernel(features, edge_index, edge_weight, W0, b0, W1, b1, W2, b2):
    src = edge_index[0]
    dst = edge_index[1]
    pad = ((0, 0), (0, EPAD - EP))
    srcp = jnp.pad(src.reshape(NW, EP), pad).reshape(NW, K, C)
    padvals = N + (jnp.arange(EPAD - EP, dtype=jnp.int32) % (NPAD - N))
    dstp = jnp.pad(dst.reshape(NW, EP), pad)
    dstp = dstp.at[:, EP:].set(padvals[None, :]).reshape(NW, K, C)
    wbits = jax.lax.bitcast_convert_type(
        jnp.pad(edge_weight.reshape(NW, EP), pad), jnp.int32).reshape(NW, K, C)
    idxw = jnp.stack([srcp, dstp, wbits], axis=2)

    spmm128 = _make_spmm(F_IN)

    agg0 = spmm128(features, idxw)
    h0a, h0b = _layer0_mm(agg0, W0, b0.reshape(1, F_HID))
    o1a = spmm128(h0a, idxw)
    o1b = spmm128(h0b, idxw)
    y2 = _layer12_mm(o1a, o1b, W1, b1.reshape(1, F_HID), W2)
    o2 = spmm128(y2, idxw)
    return _layer2_bias(o2, b2.reshape(1, F_OUT))
